# baseline (speedup 1.0000x reference)
import functools

import jax
import jax.numpy as jnp
from jax import lax
from jax.experimental import pallas as pl
from jax.experimental.pallas import tpu as pltpu
from jax.experimental.pallas import tpu_sc as plsc

NC, NS, L = 2, 16, 16
NW = NC * NS
B = 16384
D = 32
BPW = B // NW
GROUPS = BPW // L

_mesh = plsc.VectorSubcoreMesh(core_axis_name="c", subcore_axis_name="s",
                               num_cores=NC, num_subcores=NS)


@functools.partial(
    pl.kernel,
    out_type=jax.ShapeDtypeStruct((B,), jnp.float32),
    mesh=_mesh,
    scratch_types=[
        pltpu.VMEM((BPW,), jnp.int32),
        pltpu.VMEM((BPW,), jnp.int32),
        pltpu.VMEM((BPW, D), jnp.float32),
        pltpu.VMEM((BPW, D), jnp.float32),
        pltpu.VMEM((BPW,), jnp.float32),
        pltpu.SemaphoreType.DMA,
        pltpu.SemaphoreType.DMA,
    ],
    compiler_params=pltpu.CompilerParams(use_tc_tiling_on_sc=False,
                                         needs_layout_passes=False),
)
def _dot_kernel(user_hbm, game_hbm, utab_hbm, gtab_hbm, out_hbm,
                uidx, gidx, urows, grows, outv, sem_u, sem_g):
    wid = lax.axis_index("s") * NC + lax.axis_index("c")
    base = wid * BPW

    pltpu.sync_copy(user_hbm.at[pl.ds(base, BPW)], uidx)
    pltpu.sync_copy(game_hbm.at[pl.ds(base, BPW)], gidx)
    cu = pltpu.async_copy(utab_hbm.at[uidx], urows, sem_u)
    cg = pltpu.async_copy(gtab_hbm.at[gidx], grows, sem_g)
    cu.wait()
    cg.wait()

    def group_body(g, carry):
        rows = g * L + lax.iota(jnp.int32, L)
        acc = jnp.zeros((L,), jnp.float32)
        for d in range(D):
            col = jnp.full((L,), d, jnp.int32)
            vu = plsc.load_gather(urows, [rows, col])
            vg = plsc.load_gather(grows, [rows, col])
            acc = acc + vu * vg
        outv[pl.ds(g * L, L)] = acc
        return carry

    lax.fori_loop(0, GROUPS, group_body, 0)
    pltpu.sync_copy(outv, out_hbm.at[pl.ds(base, BPW)])


def kernel(user, game, user_table, game_table):
    return _dot_kernel(user.astype(jnp.int32), game.astype(jnp.int32),
                       user_table, game_table)


# speedup vs baseline: 1.5487x; 1.5487x over previous
import functools

import jax
import jax.numpy as jnp
from jax import lax
from jax.experimental import pallas as pl
from jax.experimental.pallas import tpu as pltpu
from jax.experimental.pallas import tpu_sc as plsc

NC, NS, L = 2, 16, 16
NW = NC * NS
B = 16384
D = 32
BPW = B // NW
CHUNK = 256
NCHUNK = BPW // CHUNK
CGROUPS = CHUNK // L
LAGG = 4

_mesh = plsc.VectorSubcoreMesh(core_axis_name="c", subcore_axis_name="s",
                               num_cores=NC, num_subcores=NS)


@functools.partial(
    pl.kernel,
    out_type=jax.ShapeDtypeStruct((B,), jnp.float32),
    mesh=_mesh,
    scratch_types=[
        pltpu.VMEM((BPW,), jnp.int32),
        pltpu.VMEM((BPW,), jnp.int32),
        pltpu.VMEM((CHUNK, D), jnp.float32),
        pltpu.VMEM((CHUNK, D), jnp.float32),
        pltpu.VMEM((BPW,), jnp.float32),
        pltpu.SemaphoreType.DMA,
        pltpu.SemaphoreType.DMA,
    ],
    compiler_params=pltpu.CompilerParams(use_tc_tiling_on_sc=True,
                                         needs_layout_passes=False),
)
def _dot_kernel(user_hbm, game_hbm, utab_hbm, gtab_hbm, out_hbm,
                uidx_v, gidx_v, urows, grows, outv, sem_u, sem_g):
    wid = lax.axis_index("s") * NC + lax.axis_index("c")
    base = wid * BPW

    pltpu.sync_copy(user_hbm.at[pl.ds(base, BPW)], uidx_v)
    pltpu.sync_copy(game_hbm.at[pl.ds(base, BPW)], gidx_v)

    lane = lax.iota(jnp.int32, L)

    for c in range(NCHUNK):
        cbase = c * CHUNK

        def fire_body(g, carry):
            @pl.when(g < CGROUPS)
            def _():
                u16 = uidx_v[pl.ds(cbase + g * L, L)]
                g16 = gidx_v[pl.ds(cbase + g * L, L)]
                for j in range(L):
                    sel = lane == j
                    ru = jnp.sum(jnp.where(sel, u16, 0))
                    rg = jnp.sum(jnp.where(sel, g16, 0))
                    row = g * L + j
                    pltpu.async_copy(utab_hbm.at[pl.ds(ru, 1), :],
                                     urows.at[pl.ds(row, 1), :], sem_u)
                    pltpu.async_copy(gtab_hbm.at[pl.ds(rg, 1), :],
                                     grows.at[pl.ds(row, 1), :], sem_g)

            @pl.when(g >= LAGG)
            def _():
                for _j in range(L):
                    pltpu.make_async_copy(utab_hbm.at[pl.ds(0, 1), :],
                                          urows.at[pl.ds(0, 1), :],
                                          sem_u).wait()
                    pltpu.make_async_copy(gtab_hbm.at[pl.ds(0, 1), :],
                                          grows.at[pl.ds(0, 1), :],
                                          sem_g).wait()

            return carry

        lax.fori_loop(0, CGROUPS + LAGG, fire_body, 0)

        def group_body(g, carry):
            rows = g * L + lane
            acc = jnp.zeros((L,), jnp.float32)
            for d in range(D):
                col = jnp.full((L,), d, jnp.int32)
                vu = plsc.load_gather(urows, [rows, col])
                vg = plsc.load_gather(grows, [rows, col])
                acc = acc + vu * vg
            outv[pl.ds(cbase + g * L, L)] = acc
            return carry

        lax.fori_loop(0, CGROUPS, group_body, 0)

    pltpu.sync_copy(outv, out_hbm.at[pl.ds(base, BPW)])


def kernel(user, game, user_table, game_table):
    return _dot_kernel(user.astype(jnp.int32), game.astype(jnp.int32),
                       user_table, game_table)
